# baseline (speedup 1.0000x reference)
import functools

import jax
import jax.numpy as jnp
from jax import lax
from jax.experimental import pallas as pl
from jax.experimental.pallas import tpu as pltpu
from jax.experimental.pallas import tpu_sc as plsc

N = 10000
E = 320000
D = 128
NB = 7
NCLS = 13
FUS = D * NB

NWORK = 32
FPT = D // NWORK
CH = 6400
NCHUNK = E // CH
GPC = CH // 16
NEG = float("-inf")


DETP = 2048
DROW = DETP + 16
UNROLL = 4


def _segmax_body(xf, ef, out, xs, acc, sbuf0, dbuf0, sbuf1, dbuf1,
                 lbs, lbd, det, sem0, sem1):
    c = lax.axis_index("c")
    s = lax.axis_index("s")
    w = s * 2 + c
    f0 = w * FPT

    sb = [sbuf0, sbuf1]
    db = [dbuf0, dbuf1]
    sems = [sem0, sem1]

    def chunk_copies(ci, k):
        return (pltpu.make_async_copy(ef.at[pl.ds(ci * CH, CH)], sb[k],
                                      sems[k]),
                pltpu.make_async_copy(ef.at[pl.ds(E + ci * CH, CH)], db[k],
                                      sems[k]))

    for cp in chunk_copies(0, 0):
        cp.start()
    for cp in chunk_copies(1, 1):
        cp.start()

    pltpu.sync_copy(xf.at[pl.ds(f0 * N, FPT * N)], xs)

    neg = jnp.full((16,), NEG, jnp.float32)

    def init_body(i, _):
        acc[pl.ds(i * 16, 16)] = neg
        return 0

    lax.fori_loop(0, FPT * N // 16, init_body, 0)

    lane = lax.iota(jnp.int32, 16)
    offsN = [jnp.full((16,), f * N, jnp.int32) for f in range(FPT)]

    def update(srcv, dstv, accv, det_base, off):
        xg = [plsc.load_gather(xs, [srcv + offsN[f]]) for f in range(FPT)]
        valid = dstv < N
        dix = jnp.where(valid, dstv & (DETP - 1), lane + DETP)
        dix = dix + jnp.full((16,), det_base, jnp.int32)
        plsc.store_scatter(det, [dix], lane)
        rb = plsc.load_gather(det, [dix])
        win = rb == lane
        wm = win & valid
        lm = (~win) & valid
        for f in range(FPT):
            aix = accv + offsN[f]
            a = plsc.load_gather(acc, [aix])
            plsc.store_scatter(acc, [aix], jnp.maximum(a, xg[f]), mask=wm)
        cnt = plsc.all_reduce_population_count(lm)[0]
        plsc.store_compressed(lbs.at[pl.ds(off, 16)], srcv, mask=lm)
        plsc.store_compressed(lbd.at[pl.ds(off, 16)], dstv, mask=lm)
        return off + cnt

    def do_chunk(ci, k):
        for cp in chunk_copies(ci, k):
            cp.wait()

        sbuf = sb[k]
        dbuf = db[k]

        def group(g, off):
            for u in range(UNROLL):
                b = g * (16 * UNROLL) + u * 16
                srcv = sbuf[pl.ds(b, 16)]
                dstv = dbuf[pl.ds(b, 16)]
                off = update(srcv, dstv, dstv, u * DROW, off)
            return off

        nlost = lax.fori_loop(0, GPC // UNROLL, group, 0)

        def drain_cond(L):
            return L > 0

        def drain_pass(L):
            def gbody(g, off):
                base = g * 16
                srcv = lbs[pl.ds(base, 16)]
                dstv = lbd[pl.ds(base, 16)]
                valid = lane < (L - base)
                dstv = jnp.where(valid, dstv, N + lane)
                srcv = jnp.where(valid, srcv, 0)
                accv = jnp.where(valid, dstv, 0)
                return update(srcv, dstv, accv, 0, off)

            ng = (L + 15) // 16
            return lax.fori_loop(0, ng, gbody, 0)

        lax.while_loop(drain_cond, drain_pass, nlost)

        @pl.when(ci + 2 < NCHUNK)
        def _():
            for cp in chunk_copies(ci + 2, k):
                cp.start()

    def pair(i, _):
        do_chunk(2 * i, 0)
        do_chunk(2 * i + 1, 1)
        return 0

    lax.fori_loop(0, NCHUNK // 2, pair, 0)

    pltpu.sync_copy(acc, out.at[pl.ds(f0 * N, FPT * N)])


@functools.partial(
    pl.kernel,
    out_type=jax.ShapeDtypeStruct((D * N,), jnp.float32),
    mesh=plsc.VectorSubcoreMesh(core_axis_name="c", subcore_axis_name="s"),
    compiler_params=pltpu.CompilerParams(needs_layout_passes=False),
    scratch_types=[
        pltpu.VMEM((FPT * N,), jnp.float32),
        pltpu.VMEM((FPT * N,), jnp.float32),
        pltpu.VMEM((CH,), jnp.int32),
        pltpu.VMEM((CH,), jnp.int32),
        pltpu.VMEM((CH,), jnp.int32),
        pltpu.VMEM((CH,), jnp.int32),
        pltpu.VMEM((CH + 16,), jnp.int32),
        pltpu.VMEM((CH + 16,), jnp.int32),
        pltpu.VMEM((UNROLL * DROW,), jnp.int32),
        pltpu.SemaphoreType.DMA,
        pltpu.SemaphoreType.DMA,
    ],
)
def _segmax_flat(xf, ef, out, xs, acc, sbuf0, dbuf0, sbuf1, dbuf1,
                 lbs, lbd, det, sem0, sem1):
    _segmax_body(xf, ef, out, xs, acc, sbuf0, dbuf0, sbuf1, dbuf1,
                 lbs, lbd, det, sem0, sem1)


def _segmax(xT, eflat):
    return _segmax_flat(xT.reshape(-1), eflat).reshape(D, N)


CT = 2048


def _block_body(res, xT_ref, sg_ref, wt_ref, b_ref, o_ref):
    x = xT_ref[...]
    sg = sg_ref[...]
    agg = jnp.where(sg > NEG, sg - x, 0.0)
    cat = jnp.concatenate([x, agg], axis=0)
    y = jnp.dot(wt_ref[...], cat, preferred_element_type=jnp.float32)
    y = jnp.maximum(y + b_ref[...], 0.0)
    if res:
        y = y + x
    o_ref[...] = y


def _block(xT, sg, Wt, b2, res):
    return pl.pallas_call(
        functools.partial(_block_body, res),
        grid=(pl.cdiv(N, CT),),
        in_specs=[
            pl.BlockSpec((D, CT), lambda i: (0, i)),
            pl.BlockSpec((D, CT), lambda i: (0, i)),
            pl.BlockSpec((D, 2 * D), lambda i: (0, 0)),
            pl.BlockSpec((D, 1), lambda i: (0, 0)),
        ],
        out_specs=pl.BlockSpec((D, CT), lambda i: (0, i)),
        out_shape=jax.ShapeDtypeStruct((D, N), jnp.float32),
    )(xT, sg, Wt, b2)


CTH = 1024


def _head_body(f1, f2, f3, f4, f5, f6, f7, wfus, bfus, wp1a, wp1b, bp1,
               wp2, bp2, wp3, bp3, o_ref):
    feats = jnp.concatenate(
        [f1[...], f2[...], f3[...], f4[...], f5[...], f6[...], f7[...]],
        axis=0)
    t = jnp.dot(wfus[...], feats, preferred_element_type=jnp.float32)
    fus = jnp.max(jnp.maximum(t + bfus[...], 0.0), axis=0, keepdims=True)
    h = jnp.dot(wp1a[...], feats, preferred_element_type=jnp.float32)
    h = jnp.maximum(h + wp1b[...] * fus + bp1[...], 0.0)
    h = jnp.dot(wp2[...], h, preferred_element_type=jnp.float32)
    h = jnp.maximum(h + bp2[...], 0.0)
    o_ref[...] = jnp.dot(wp3[...], h,
                         preferred_element_type=jnp.float32) + bp3[...]


def _head(ys, W_fus, b_fus, W_p1, b_p1, W_p2, b_p2, W_p3, b_p3):
    col = lambda i: (0, i)
    fix = lambda i: (0, 0)
    return pl.pallas_call(
        _head_body,
        grid=(pl.cdiv(N, CTH),),
        in_specs=[pl.BlockSpec((D, CTH), col)] * NB + [
            pl.BlockSpec((1024, FUS), fix),
            pl.BlockSpec((1024, 1), fix),
            pl.BlockSpec((512, FUS), fix),
            pl.BlockSpec((512, 1), fix),
            pl.BlockSpec((512, 1), fix),
            pl.BlockSpec((256, 512), fix),
            pl.BlockSpec((256, 1), fix),
            pl.BlockSpec((NCLS, 256), fix),
            pl.BlockSpec((NCLS, 1), fix),
        ],
        out_specs=pl.BlockSpec((NCLS, CTH), col),
        out_shape=jax.ShapeDtypeStruct((NCLS, N), jnp.float32),
    )(*ys, W_fus.T, b_fus[:, None], W_p1[:FUS].T, W_p1[FUS:].T,
      b_p1[:, None], W_p2.T, b_p2[:, None], W_p3.T, b_p3[:, None])


def kernel(x, edge_index, batch, W_head, b_head, W_blocks, b_blocks,
           W_fus, b_fus, W_p1, b_p1, W_p2, b_p2, W_p3, b_p3):
    xT = x.T
    eflat = edge_index.reshape(-1)
    sg = _segmax(xT, eflat)
    ys = [_block(xT, sg, W_head.T, b_head[:, None], res=False)]
    for i in range(NB - 1):
        sg = _segmax(ys[-1], eflat)
        ys.append(_block(ys[-1], sg, W_blocks[i].T, b_blocks[i][:, None],
                         res=True))
    outT = _head(ys, W_fus, b_fus, W_p1, b_p1, W_p2, b_p2, W_p3, b_p3)
    return outT.T


# speedup vs baseline: 1.0880x; 1.0880x over previous
import functools

import jax
import jax.numpy as jnp
from jax import lax
from jax.experimental import pallas as pl
from jax.experimental.pallas import tpu as pltpu
from jax.experimental.pallas import tpu_sc as plsc

N = 10000
E = 320000
D = 128
NB = 7
NCLS = 13
FUS = D * NB

NWORK = 32
FPT = D // NWORK
CH = 6400
NCHUNK = E // CH
GPC = CH // 16
NEG = float("-inf")


DETP = 2048
DROW = DETP + 16
UNROLL = 2


def _segmax_body(xf, ef, out, xs, acc, sbuf0, dbuf0, sbuf1, dbuf1,
                 lbs, lbd, det, sem0, sem1):
    c = lax.axis_index("c")
    s = lax.axis_index("s")
    w = s * 2 + c
    f0 = w * FPT

    sb = [sbuf0, sbuf1]
    db = [dbuf0, dbuf1]
    sems = [sem0, sem1]

    def chunk_copies(ci, k):
        return (pltpu.make_async_copy(ef.at[pl.ds(ci * CH, CH)], sb[k],
                                      sems[k]),
                pltpu.make_async_copy(ef.at[pl.ds(E + ci * CH, CH)], db[k],
                                      sems[k]))

    for cp in chunk_copies(0, 0):
        cp.start()
    for cp in chunk_copies(1, 1):
        cp.start()

    pltpu.sync_copy(xf.at[pl.ds(f0 * N, FPT * N)], xs)

    neg = jnp.full((16,), NEG, jnp.float32)

    def init_body(i, _):
        acc[pl.ds(i * 16, 16)] = neg
        return 0

    lax.fori_loop(0, FPT * N // 16, init_body, 0)

    lane = lax.iota(jnp.int32, 16)
    offsN = [jnp.full((16,), f * N, jnp.int32) for f in range(FPT)]

    def update(srcv, dstv, accv, det_base, off, sentinels):
        xg = [plsc.load_gather(xs, [srcv + offsN[f]]) for f in range(FPT)]
        if sentinels:
            valid = dstv < N
            dix = jnp.where(valid, dstv & (DETP - 1), lane + DETP)
        else:
            dix = dstv & (DETP - 1)
        dix = dix + jnp.full((16,), det_base, jnp.int32)
        plsc.store_scatter(det, [dix], lane)
        rb = plsc.load_gather(det, [dix])
        win = rb == lane
        if sentinels:
            wm = win & valid
            lm = (~win) & valid
        else:
            wm = win
            lm = ~win
        for f in range(FPT):
            aix = accv + offsN[f]
            a = plsc.load_gather(acc, [aix])
            plsc.store_scatter(acc, [aix], jnp.maximum(a, xg[f]), mask=wm)
        cnt = plsc.all_reduce_population_count(lm)[0]

        @pl.when(cnt > 0)
        def _():
            plsc.store_compressed(lbs.at[pl.ds(off, 16)], srcv, mask=lm)
            plsc.store_compressed(lbd.at[pl.ds(off, 16)], dstv, mask=lm)

        return off + cnt

    def do_chunk(ci, k):
        for cp in chunk_copies(ci, k):
            cp.wait()

        sbuf = sb[k]
        dbuf = db[k]

        def group(g, off):
            for u in range(UNROLL):
                b = g * (16 * UNROLL) + u * 16
                srcv = sbuf[pl.ds(b, 16)]
                dstv = dbuf[pl.ds(b, 16)]
                off = update(srcv, dstv, dstv, u * DROW, off, False)
            return off

        nlost = lax.fori_loop(0, GPC // UNROLL, group, 0)

        def drain_cond(L):
            return L > 0

        def drain_pass(L):
            def gbody(g, off):
                base = g * 16
                srcv = lbs[pl.ds(base, 16)]
                dstv = lbd[pl.ds(base, 16)]
                valid = lane < (L - base)
                dstv = jnp.where(valid, dstv, N + lane)
                srcv = jnp.where(valid, srcv, 0)
                accv = jnp.where(valid, dstv, 0)
                return update(srcv, dstv, accv, 0, off, True)

            ng = (L + 15) // 16
            return lax.fori_loop(0, ng, gbody, 0)

        lax.while_loop(drain_cond, drain_pass, nlost)

        @pl.when(ci + 2 < NCHUNK)
        def _():
            for cp in chunk_copies(ci + 2, k):
                cp.start()

    def pair(i, _):
        do_chunk(2 * i, 0)
        do_chunk(2 * i + 1, 1)
        return 0

    lax.fori_loop(0, NCHUNK // 2, pair, 0)

    pltpu.sync_copy(acc, out.at[pl.ds(f0 * N, FPT * N)])


@functools.partial(
    pl.kernel,
    out_type=jax.ShapeDtypeStruct((D * N,), jnp.float32),
    mesh=plsc.VectorSubcoreMesh(core_axis_name="c", subcore_axis_name="s"),
    compiler_params=pltpu.CompilerParams(needs_layout_passes=False),
    scratch_types=[
        pltpu.VMEM((FPT * N,), jnp.float32),
        pltpu.VMEM((FPT * N,), jnp.float32),
        pltpu.VMEM((CH,), jnp.int32),
        pltpu.VMEM((CH,), jnp.int32),
        pltpu.VMEM((CH,), jnp.int32),
        pltpu.VMEM((CH,), jnp.int32),
        pltpu.VMEM((CH + 16,), jnp.int32),
        pltpu.VMEM((CH + 16,), jnp.int32),
        pltpu.VMEM((UNROLL * DROW,), jnp.int32),
        pltpu.SemaphoreType.DMA,
        pltpu.SemaphoreType.DMA,
    ],
)
def _segmax_flat(xf, ef, out, xs, acc, sbuf0, dbuf0, sbuf1, dbuf1,
                 lbs, lbd, det, sem0, sem1):
    _segmax_body(xf, ef, out, xs, acc, sbuf0, dbuf0, sbuf1, dbuf1,
                 lbs, lbd, det, sem0, sem1)


def _segmax(xT, eflat):
    return _segmax_flat(xT.reshape(-1), eflat).reshape(D, N)


CT = 2048


def _block_body(res, xT_ref, sg_ref, wt_ref, b_ref, o_ref):
    x = xT_ref[...]
    sg = sg_ref[...]
    agg = jnp.where(sg > NEG, sg - x, 0.0)
    cat = jnp.concatenate([x, agg], axis=0)
    y = jnp.dot(wt_ref[...], cat, preferred_element_type=jnp.float32)
    y = jnp.maximum(y + b_ref[...], 0.0)
    if res:
        y = y + x
    o_ref[...] = y


def _block(xT, sg, Wt, b2, res):
    return pl.pallas_call(
        functools.partial(_block_body, res),
        grid=(pl.cdiv(N, CT),),
        in_specs=[
            pl.BlockSpec((D, CT), lambda i: (0, i)),
            pl.BlockSpec((D, CT), lambda i: (0, i)),
            pl.BlockSpec((D, 2 * D), lambda i: (0, 0)),
            pl.BlockSpec((D, 1), lambda i: (0, 0)),
        ],
        out_specs=pl.BlockSpec((D, CT), lambda i: (0, i)),
        out_shape=jax.ShapeDtypeStruct((D, N), jnp.float32),
    )(xT, sg, Wt, b2)


CTH = 1024


def _head_body(f1, f2, f3, f4, f5, f6, f7, wfus, bfus, wp1a, wp1b, bp1,
               wp2, bp2, wp3, bp3, o_ref):
    feats = jnp.concatenate(
        [f1[...], f2[...], f3[...], f4[...], f5[...], f6[...], f7[...]],
        axis=0)
    t = jnp.dot(wfus[...], feats, preferred_element_type=jnp.float32)
    fus = jnp.max(jnp.maximum(t + bfus[...], 0.0), axis=0, keepdims=True)
    h = jnp.dot(wp1a[...], feats, preferred_element_type=jnp.float32)
    h = jnp.maximum(h + wp1b[...] * fus + bp1[...], 0.0)
    h = jnp.dot(wp2[...], h, preferred_element_type=jnp.float32)
    h = jnp.maximum(h + bp2[...], 0.0)
    o_ref[...] = jnp.dot(wp3[...], h,
                         preferred_element_type=jnp.float32) + bp3[...]


def _head(ys, W_fus, b_fus, W_p1, b_p1, W_p2, b_p2, W_p3, b_p3):
    col = lambda i: (0, i)
    fix = lambda i: (0, 0)
    return pl.pallas_call(
        _head_body,
        grid=(pl.cdiv(N, CTH),),
        in_specs=[pl.BlockSpec((D, CTH), col)] * NB + [
            pl.BlockSpec((1024, FUS), fix),
            pl.BlockSpec((1024, 1), fix),
            pl.BlockSpec((512, FUS), fix),
            pl.BlockSpec((512, 1), fix),
            pl.BlockSpec((512, 1), fix),
            pl.BlockSpec((256, 512), fix),
            pl.BlockSpec((256, 1), fix),
            pl.BlockSpec((NCLS, 256), fix),
            pl.BlockSpec((NCLS, 1), fix),
        ],
        out_specs=pl.BlockSpec((NCLS, CTH), col),
        out_shape=jax.ShapeDtypeStruct((NCLS, N), jnp.float32),
    )(*ys, W_fus.T, b_fus[:, None], W_p1[:FUS].T, W_p1[FUS:].T,
      b_p1[:, None], W_p2.T, b_p2[:, None], W_p3.T, b_p3[:, None])


def kernel(x, edge_index, batch, W_head, b_head, W_blocks, b_blocks,
           W_fus, b_fus, W_p1, b_p1, W_p2, b_p2, W_p3, b_p3):
    xT = x.T
    eflat = edge_index.reshape(-1)
    sg = _segmax(xT, eflat)
    ys = [_block(xT, sg, W_head.T, b_head[:, None], res=False)]
    for i in range(NB - 1):
        sg = _segmax(ys[-1], eflat)
        ys.append(_block(ys[-1], sg, W_blocks[i].T, b_blocks[i][:, None],
                         res=True))
    outT = _head(ys, W_fus, b_fus, W_p1, b_p1, W_p2, b_p2, W_p3, b_p3)
    return outT.T
